# baseline (speedup 1.0000x reference)
import functools

import jax
import jax.numpy as jnp
from jax import lax
from jax.experimental import pallas as pl
from jax.experimental.pallas import tpu as pltpu
from jax.experimental.pallas import tpu_sc as plsc

N = 10000
D = 128
E = 320000

NC = 2
NS = 16
NW = NC * NS
CHUNK = 128
CH = 80
IB = 40
EPAD = NW * CH * CHUNK
NPAD = 10240
RPS = NPAD // NS

BLK = 1280
GRID = NPAD // BLK

_mesh = plsc.VectorSubcoreMesh(core_axis_name="c", subcore_axis_name="s")
_f32 = jnp.float32


def _fill(ref, value):
    @pl.loop(0, CHUNK)
    def _(i):
        @pl.loop(0, D, step=16)
        def _(c):
            ref.at[i, pl.ds(c, 16)][...] = jnp.full((16,), value, _f32)


DW = 16
DK = 8


@functools.partial(
    pl.kernel,
    out_type=jax.ShapeDtypeStruct((NC, NPAD, D), _f32),
    mesh=_mesh,
    scratch_types=[
        pltpu.VMEM((CH, CHUNK), jnp.int32),
        pltpu.VMEM((CHUNK, D), _f32),
        pltpu.VMEM((CHUNK, D), _f32),
        pltpu.VMEM_SHARED((NPAD, D), _f32),
        pltpu.SemaphoreType.DMA,
    ],
)
def _sc_degree(dst_hbm, out_hbm, dst_v, ones_v, zero_v, acc_sh, sem):
    cid = lax.axis_index("c")
    sid = lax.axis_index("s")
    wid = sid * NC + cid

    _fill(ones_v, 1.0)
    _fill(zero_v, 0.0)

    @pl.loop(0, RPS // CHUNK)
    def _(k):
        pltpu.sync_copy(zero_v, acc_sh.at[pl.ds(sid * RPS + k * CHUNK, CHUNK)])

    plsc.subcore_barrier()
    pltpu.sync_copy(dst_hbm.at[wid], dst_v)

    @pl.loop(0, CH, step=DK)
    def _(j):
        for k in range(DK):
            pltpu.async_copy(ones_v, acc_sh.at[dst_v.at[j + k]], sem,
                             add=True)
        for k in range(DK):
            pltpu.make_async_copy(ones_v, acc_sh.at[dst_v.at[j + k]],
                                  sem).wait()

    plsc.subcore_barrier()
    pltpu.sync_copy(acc_sh.at[pl.ds(sid * RPS, RPS)],
                    out_hbm.at[cid, pl.ds(sid * RPS, RPS)])


@functools.partial(
    pl.kernel,
    out_type=jax.ShapeDtypeStruct((NC, NPAD, D), _f32),
    mesh=_mesh,
    scratch_types=[
        pltpu.VMEM((IB, CHUNK), jnp.int32),
        pltpu.VMEM((IB, CHUNK), jnp.int32),
        pltpu.VMEM((CHUNK, D), _f32),
        pltpu.VMEM((CHUNK, D), _f32),
        pltpu.VMEM_SHARED((NPAD, D), _f32),
        pltpu.SemaphoreType.DMA,
        pltpu.SemaphoreType.DMA,
        pltpu.SemaphoreType.DMA,
        pltpu.SemaphoreType.DMA,
    ],
)
def _sc_edge_pass(g_hbm, src_hbm, dst_hbm, out_hbm,
                  src_i, dst_i, rows0, rows1, acc_sh, sem0, sem1,
                  sem_s0, sem_s1):
    cid = lax.axis_index("c")
    sid = lax.axis_index("s")
    wid = sid * NC + cid

    _fill(rows0, 0.0)

    @pl.loop(0, RPS // CHUNK)
    def _(k):
        pltpu.sync_copy(rows0, acc_sh.at[pl.ds(sid * RPS + k * CHUNK, CHUNK)])

    plsc.subcore_barrier()

    @pl.loop(0, CH, step=IB)
    def _(jb):
        pltpu.sync_copy(src_hbm.at[wid, pl.ds(jb, IB)], src_i)
        pltpu.sync_copy(dst_hbm.at[wid, pl.ds(jb, IB)], dst_i)
        pltpu.async_copy(g_hbm.at[src_i.at[0]], rows0, sem0)
        pltpu.async_copy(g_hbm.at[src_i.at[1]], rows1, sem1)

        @pl.loop(0, IB, step=2)
        def _(t):
            pltpu.make_async_copy(g_hbm.at[src_i.at[t]], rows0, sem0).wait()
            pltpu.async_copy(rows0, acc_sh.at[dst_i.at[t]], sem_s0, add=True)

            pltpu.make_async_copy(g_hbm.at[src_i.at[t + 1]], rows1, sem1).wait()
            pltpu.async_copy(rows1, acc_sh.at[dst_i.at[t + 1]], sem_s1,
                             add=True)

            @pl.when(t + 2 < IB)
            def _():
                pltpu.make_async_copy(rows0, acc_sh.at[dst_i.at[t]],
                                      sem_s0).wait()
                pltpu.async_copy(g_hbm.at[src_i.at[t + 2]], rows0, sem0)

            @pl.when(t + 3 < IB)
            def _():
                pltpu.make_async_copy(rows1, acc_sh.at[dst_i.at[t + 1]],
                                      sem_s1).wait()
                pltpu.async_copy(g_hbm.at[src_i.at[t + 3]], rows1, sem1)

        pltpu.make_async_copy(rows0, acc_sh.at[dst_i.at[IB - 2]],
                              sem_s0).wait()
        pltpu.make_async_copy(rows1, acc_sh.at[dst_i.at[IB - 1]],
                              sem_s1).wait()

    plsc.subcore_barrier()
    pltpu.sync_copy(acc_sh.at[pl.ds(sid * RPS, RPS)],
                    out_hbm.at[cid, pl.ds(sid * RPS, RPS)])


def _dinv_block(dega_ref, degb_ref):
    deg = dega_ref[:, 0:1] + degb_ref[:, 0:1] + 1.0
    return lax.rsqrt(deg)


def _mm_body(x_ref, w_ref, o_ref):
    o_ref[...] = jnp.dot(x_ref[...], w_ref[...],
                         preferred_element_type=_f32,
                         precision=lax.Precision.HIGHEST)


def _scale_body(h_ref, dega_ref, degb_ref, o_ref):
    o_ref[...] = h_ref[...] * _dinv_block(dega_ref, degb_ref)


def _mid_body(g_ref, acca_ref, accb_ref, dega_ref, degb_ref, w_ref, b_ref,
              o_ref):
    dinv = _dinv_block(dega_ref, degb_ref)
    h = (g_ref[...] + acca_ref[...] + accb_ref[...]) * dinv + b_ref[...]
    h = jnp.maximum(h, 0.0)
    o_ref[...] = jnp.dot(h, w_ref[...], preferred_element_type=_f32,
                         precision=lax.Precision.HIGHEST) * dinv


def _final_body(g_ref, acca_ref, accb_ref, dega_ref, degb_ref, b_ref, o_ref):
    dinv = _dinv_block(dega_ref, degb_ref)
    o_ref[...] = (g_ref[...] + acca_ref[...] + accb_ref[...]) * dinv + b_ref[...]


_row_spec = pl.BlockSpec((BLK, D), lambda i: (i, 0))
_deg_spec = pl.BlockSpec((BLK, DW), lambda i: (i, 0))
_w_spec = pl.BlockSpec((D, D), lambda i: (0, 0))
_b_spec = pl.BlockSpec((1, D), lambda i: (0, 0))
_row_out = jax.ShapeDtypeStruct((NPAD, D), _f32)

_tc_mm = pl.pallas_call(
    _mm_body, grid=(GRID,), out_shape=_row_out,
    in_specs=[_row_spec, _w_spec], out_specs=_row_spec)

_tc_scale = pl.pallas_call(
    _scale_body, grid=(GRID,), out_shape=_row_out,
    in_specs=[_row_spec, _deg_spec, _deg_spec], out_specs=_row_spec)

_tc_mid = pl.pallas_call(
    _mid_body, grid=(GRID,), out_shape=_row_out,
    in_specs=[_row_spec, _row_spec, _row_spec, _deg_spec, _deg_spec,
              _w_spec, _b_spec],
    out_specs=_row_spec)

_tc_final = pl.pallas_call(
    _final_body, grid=(GRID,), out_shape=_row_out,
    in_specs=[_row_spec, _row_spec, _row_spec, _deg_spec, _deg_spec, _b_spec],
    out_specs=_row_spec)


def kernel(x, edge_index, W1, b1, W2, b2):
    src = edge_index[0].astype(jnp.int32)
    dst = edge_index[1].astype(jnp.int32)
    pad = N + jnp.arange(EPAD - E, dtype=jnp.int32) % (NPAD - N)
    src_r = jnp.concatenate([src, pad]).reshape(NW, CH, CHUNK)
    dst_r = jnp.concatenate([dst, pad]).reshape(NW, CH, CHUNK)
    x_pad = jnp.pad(x, ((0, NPAD - N), (0, 0)))
    b1r = b1.reshape(1, D)
    b2r = b2.reshape(1, D)

    deg = _sc_degree(dst_r)
    dega, degb = deg[0, :, :DW], deg[1, :, :DW]

    h1 = _tc_mm(x_pad, W1)
    g1 = _tc_scale(h1, dega, degb)
    acc1 = _sc_edge_pass(g1, src_r, dst_r)
    g2 = _tc_mid(g1, acc1[0], acc1[1], dega, degb, W2, b1r)
    acc2 = _sc_edge_pass(g2, src_r, dst_r)
    out = _tc_final(g2, acc2[0], acc2[1], dega, degb, b2r)
    return out[:N]


# speedup vs baseline: 1.2156x; 1.2156x over previous
import functools

import jax
import jax.numpy as jnp
from jax import lax
from jax.experimental import pallas as pl
from jax.experimental.pallas import tpu as pltpu
from jax.experimental.pallas import tpu_sc as plsc

N = 10000
D = 128
E = 320000

NC = 2
NS = 16
NW = NC * NS
CHUNK = 128
CH = 80
IB = 40
EPAD = NW * CH * CHUNK
NPAD = 10240
RPS = NPAD // NS

BLK = 1280
GRID = NPAD // BLK

_mesh = plsc.VectorSubcoreMesh(core_axis_name="c", subcore_axis_name="s")
_f32 = jnp.float32


def _fill(ref, value):
    @pl.loop(0, CHUNK)
    def _(i):
        @pl.loop(0, D, step=16)
        def _(c):
            ref.at[i, pl.ds(c, 16)][...] = jnp.full((16,), value, _f32)


DW = 16
DK = 8


@functools.partial(
    pl.kernel,
    out_type=jax.ShapeDtypeStruct((NC, NPAD, D), _f32),
    mesh=_mesh,
    scratch_types=[
        pltpu.VMEM((CH, CHUNK), jnp.int32),
        pltpu.VMEM((CHUNK, D), _f32),
        pltpu.VMEM((CHUNK, D), _f32),
        pltpu.VMEM_SHARED((NPAD, D), _f32),
        pltpu.SemaphoreType.DMA,
    ],
)
def _sc_degree(dst_hbm, out_hbm, dst_v, ones_v, zero_v, acc_sh, sem):
    cid = lax.axis_index("c")
    sid = lax.axis_index("s")
    wid = sid * NC + cid

    _fill(ones_v, 1.0)
    _fill(zero_v, 0.0)

    @pl.loop(0, RPS // CHUNK)
    def _(k):
        pltpu.sync_copy(zero_v, acc_sh.at[pl.ds(sid * RPS + k * CHUNK, CHUNK)])

    plsc.subcore_barrier()
    pltpu.sync_copy(dst_hbm.at[wid], dst_v)

    @pl.loop(0, CH, step=DK)
    def _(j):
        for k in range(DK):
            pltpu.async_copy(ones_v, acc_sh.at[dst_v.at[j + k]], sem,
                             add=True)
        for k in range(DK):
            pltpu.make_async_copy(ones_v, acc_sh.at[dst_v.at[j + k]],
                                  sem).wait()

    plsc.subcore_barrier()
    pltpu.sync_copy(acc_sh.at[pl.ds(sid * RPS, RPS)],
                    out_hbm.at[cid, pl.ds(sid * RPS, RPS)])


@functools.partial(
    pl.kernel,
    out_type=jax.ShapeDtypeStruct((NC, NPAD, D), _f32),
    mesh=_mesh,
    scratch_types=[
        pltpu.VMEM((IB, CHUNK), jnp.int32),
        pltpu.VMEM((IB, CHUNK), jnp.int32),
        pltpu.VMEM((CHUNK, D), _f32),
        pltpu.VMEM((CHUNK, D), _f32),
        pltpu.VMEM_SHARED((NPAD, D), _f32),
        pltpu.SemaphoreType.DMA,
        pltpu.SemaphoreType.DMA,
    ],
)
def _sc_edge_pass(g_hbm, src_hbm, dst_hbm, out_hbm,
                  src_i, dst_i, rows0, rows1, acc_sh, sem0, sem1):
    cid = lax.axis_index("c")
    sid = lax.axis_index("s")
    wid = sid * NC + cid

    _fill(rows0, 0.0)

    @pl.loop(0, RPS // CHUNK)
    def _(k):
        pltpu.sync_copy(rows0, acc_sh.at[pl.ds(sid * RPS + k * CHUNK, CHUNK)])

    plsc.subcore_barrier()

    @pl.loop(0, CH, step=IB)
    def _(jb):
        pltpu.sync_copy(src_hbm.at[wid, pl.ds(jb, IB)], src_i)
        pltpu.sync_copy(dst_hbm.at[wid, pl.ds(jb, IB)], dst_i)
        pltpu.async_copy(g_hbm.at[src_i.at[0]], rows0, sem0)
        pltpu.async_copy(g_hbm.at[src_i.at[1]], rows1, sem1)

        @pl.loop(0, IB, step=2)
        def _(t):
            pltpu.make_async_copy(g_hbm.at[src_i.at[t]], rows0, sem0).wait()
            pltpu.sync_copy(rows0, acc_sh.at[dst_i.at[t]], add=True)

            @pl.when(t + 2 < IB)
            def _():
                pltpu.async_copy(g_hbm.at[src_i.at[t + 2]], rows0, sem0)

            pltpu.make_async_copy(g_hbm.at[src_i.at[t + 1]], rows1, sem1).wait()
            pltpu.sync_copy(rows1, acc_sh.at[dst_i.at[t + 1]], add=True)

            @pl.when(t + 3 < IB)
            def _():
                pltpu.async_copy(g_hbm.at[src_i.at[t + 3]], rows1, sem1)

    plsc.subcore_barrier()
    pltpu.sync_copy(acc_sh.at[pl.ds(sid * RPS, RPS)],
                    out_hbm.at[cid, pl.ds(sid * RPS, RPS)])


def _dinv_block(dega_ref, degb_ref):
    deg = dega_ref[:, 0:1] + degb_ref[:, 0:1] + 1.0
    return lax.rsqrt(deg)


def _mm_body(x_ref, w_ref, o_ref):
    o_ref[...] = jnp.dot(x_ref[...], w_ref[...],
                         preferred_element_type=_f32,
                         precision=lax.Precision.HIGHEST)


def _scale_body(h_ref, dega_ref, degb_ref, o_ref):
    o_ref[...] = h_ref[...] * _dinv_block(dega_ref, degb_ref)


def _mid_body(g_ref, acca_ref, accb_ref, dega_ref, degb_ref, w_ref, b_ref,
              o_ref):
    dinv = _dinv_block(dega_ref, degb_ref)
    h = (g_ref[...] + acca_ref[...] + accb_ref[...]) * dinv + b_ref[...]
    h = jnp.maximum(h, 0.0)
    o_ref[...] = jnp.dot(h, w_ref[...], preferred_element_type=_f32,
                         precision=lax.Precision.HIGHEST) * dinv


def _final_body(g_ref, acca_ref, accb_ref, dega_ref, degb_ref, b_ref, o_ref):
    dinv = _dinv_block(dega_ref, degb_ref)
    o_ref[...] = (g_ref[...] + acca_ref[...] + accb_ref[...]) * dinv + b_ref[...]


_row_spec = pl.BlockSpec((BLK, D), lambda i: (i, 0))
_deg_spec = pl.BlockSpec((BLK, DW), lambda i: (i, 0))
_w_spec = pl.BlockSpec((D, D), lambda i: (0, 0))
_b_spec = pl.BlockSpec((1, D), lambda i: (0, 0))
_row_out = jax.ShapeDtypeStruct((NPAD, D), _f32)

FBLK = 2000
FGRID = N // FBLK
_frow_spec = pl.BlockSpec((FBLK, D), lambda i: (i, 0))
_fdeg_spec = pl.BlockSpec((FBLK, DW), lambda i: (i, 0))
_fb_spec = pl.BlockSpec((1, D), lambda i: (0, 0))

_tc_mm = pl.pallas_call(
    _mm_body, grid=(GRID,), out_shape=_row_out,
    in_specs=[_row_spec, _w_spec], out_specs=_row_spec)

_tc_scale = pl.pallas_call(
    _scale_body, grid=(GRID,), out_shape=_row_out,
    in_specs=[_row_spec, _deg_spec, _deg_spec], out_specs=_row_spec)

_tc_mid = pl.pallas_call(
    _mid_body, grid=(GRID,), out_shape=_row_out,
    in_specs=[_row_spec, _row_spec, _row_spec, _deg_spec, _deg_spec,
              _w_spec, _b_spec],
    out_specs=_row_spec)

_tc_final = pl.pallas_call(
    _final_body, grid=(FGRID,), out_shape=jax.ShapeDtypeStruct((N, D), _f32),
    in_specs=[_frow_spec, _frow_spec, _frow_spec, _fdeg_spec, _fdeg_spec,
              _fb_spec],
    out_specs=_frow_spec)


def kernel(x, edge_index, W1, b1, W2, b2):
    src = edge_index[0].astype(jnp.int32)
    dst = edge_index[1].astype(jnp.int32)
    pad = N + jnp.arange(EPAD - E, dtype=jnp.int32) % (NPAD - N)
    src_r = jnp.concatenate([src, pad]).reshape(NW, CH, CHUNK)
    dst_r = jnp.concatenate([dst, pad]).reshape(NW, CH, CHUNK)
    b1r = b1.reshape(1, D)
    b2r = b2.reshape(1, D)

    deg = _sc_degree(dst_r)
    dega, degb = deg[0, :, :DW], deg[1, :, :DW]

    h1 = _tc_mm(x, W1)
    g1 = _tc_scale(h1, dega, degb)
    acc1 = _sc_edge_pass(g1, src_r, dst_r)
    g2 = _tc_mid(g1, acc1[0], acc1[1], dega, degb, W2, b1r)
    acc2 = _sc_edge_pass(g2, src_r, dst_r)
    return _tc_final(g2, acc2[0], acc2[1], dega, degb, b2r)


# speedup vs baseline: 1.4391x; 1.1838x over previous
import dataclasses
import functools

import jax
import jax.numpy as jnp
from jax import lax
from jax.experimental import pallas as pl
from jax.experimental.pallas import tpu as pltpu
from jax.experimental.pallas import tpu_sc as plsc

N = 10000
D = 128
E = 320000

NC = 2
NS = 16
NW = NC * NS
CHUNK = 128
CH = 80
IB = 40
EPAD = NW * CH * CHUNK
NPAD = 10240
RPS = NPAD // NS

BLK = 1280
GRID = NPAD // BLK

_mesh = plsc.VectorSubcoreMesh(core_axis_name="c", subcore_axis_name="s")
_f32 = jnp.float32


def _fill(ref, value):
    @pl.loop(0, CHUNK)
    def _(i):
        @pl.loop(0, D, step=16)
        def _(c):
            ref.at[i, pl.ds(c, 16)][...] = jnp.full((16,), value, _f32)


HR = NPAD // CHUNK

_sc_params = pltpu.CompilerParams()
if "needs_layout_passes" in pltpu.CompilerParams.__dataclass_fields__:
    _sc_params = dataclasses.replace(_sc_params, needs_layout_passes=False)


@functools.partial(
    pl.kernel,
    out_type=jax.ShapeDtypeStruct((NC, HR, CHUNK), _f32),
    mesh=_mesh,
    compiler_params=_sc_params,
    scratch_types=[
        pltpu.VMEM((CH, CHUNK), jnp.int32),
        pltpu.VMEM((HR, CHUNK), _f32),
        pltpu.VMEM((8, CHUNK), _f32),
        pltpu.VMEM((HR,), jnp.int32),
        pltpu.VMEM_SHARED((HR, CHUNK), _f32),
    ],
)
def _sc_degree(dst_hbm, out_hbm, dst_v, hist_v, zero_v, iota_v, acc_sh):
    cid = lax.axis_index("c")
    sid = lax.axis_index("s")
    wid = sid * NC + cid

    lane = lax.iota(jnp.int32, 16)

    @pl.loop(0, HR)
    def _(r):
        @pl.loop(0, CHUNK, step=16)
        def _(c):
            hist_v.at[r, pl.ds(c, 16)][...] = jnp.zeros((16,), _f32)

    @pl.loop(0, 8)
    def _(r):
        @pl.loop(0, CHUNK, step=16)
        def _(c):
            zero_v.at[r, pl.ds(c, 16)][...] = jnp.zeros((16,), _f32)

    @pl.loop(0, HR, step=16)
    def _(r):
        iota_v.at[pl.ds(r, 16)][...] = lane + r

    @pl.when(sid < HR // 8)
    def _():
        pltpu.sync_copy(zero_v, acc_sh.at[pl.ds(sid * 8, 8)])

    plsc.subcore_barrier()
    pltpu.sync_copy(dst_hbm.at[wid], dst_v)

    ones = jnp.full((16,), 1.0, _f32)

    @pl.loop(0, CH)
    def _(j):
        @pl.loop(0, CHUNK, step=16)
        def _(c):
            idx = dst_v.at[j, pl.ds(c, 16)][...]
            row = lax.shift_right_logical(idx, 7)
            col = lax.bitwise_and(idx, 127)
            plsc.addupdate_scatter(hist_v, [row, col], ones)

    pltpu.sync_copy(hist_v, acc_sh.at[iota_v], add=True)
    plsc.subcore_barrier()

    @pl.when(sid < HR // 8)
    def _():
        pltpu.sync_copy(acc_sh.at[pl.ds(sid * 8, 8)],
                        out_hbm.at[cid, pl.ds(sid * 8, 8)])


@functools.partial(
    pl.kernel,
    out_type=jax.ShapeDtypeStruct((NC, NPAD, D), _f32),
    mesh=_mesh,
    scratch_types=[
        pltpu.VMEM((IB, CHUNK), jnp.int32),
        pltpu.VMEM((IB, CHUNK), jnp.int32),
        pltpu.VMEM((CHUNK, D), _f32),
        pltpu.VMEM((CHUNK, D), _f32),
        pltpu.VMEM_SHARED((NPAD, D), _f32),
        pltpu.SemaphoreType.DMA,
        pltpu.SemaphoreType.DMA,
    ],
)
def _sc_edge_pass(g_hbm, src_hbm, dst_hbm, out_hbm,
                  src_i, dst_i, rows0, rows1, acc_sh, sem0, sem1):
    cid = lax.axis_index("c")
    sid = lax.axis_index("s")
    wid = sid * NC + cid

    _fill(rows0, 0.0)

    @pl.loop(0, RPS // CHUNK)
    def _(k):
        pltpu.sync_copy(rows0, acc_sh.at[pl.ds(sid * RPS + k * CHUNK, CHUNK)])

    plsc.subcore_barrier()

    @pl.loop(0, CH, step=IB)
    def _(jb):
        pltpu.sync_copy(src_hbm.at[wid, pl.ds(jb, IB)], src_i)
        pltpu.sync_copy(dst_hbm.at[wid, pl.ds(jb, IB)], dst_i)
        pltpu.async_copy(g_hbm.at[src_i.at[0]], rows0, sem0)
        pltpu.async_copy(g_hbm.at[src_i.at[1]], rows1, sem1)

        @pl.loop(0, IB, step=2)
        def _(t):
            pltpu.make_async_copy(g_hbm.at[src_i.at[t]], rows0, sem0).wait()
            pltpu.sync_copy(rows0, acc_sh.at[dst_i.at[t]], add=True)

            @pl.when(t + 2 < IB)
            def _():
                pltpu.async_copy(g_hbm.at[src_i.at[t + 2]], rows0, sem0)

            pltpu.make_async_copy(g_hbm.at[src_i.at[t + 1]], rows1, sem1).wait()
            pltpu.sync_copy(rows1, acc_sh.at[dst_i.at[t + 1]], add=True)

            @pl.when(t + 3 < IB)
            def _():
                pltpu.async_copy(g_hbm.at[src_i.at[t + 3]], rows1, sem1)

    plsc.subcore_barrier()
    pltpu.sync_copy(acc_sh.at[pl.ds(sid * RPS, RPS)],
                    out_hbm.at[cid, pl.ds(sid * RPS, RPS)])


def _dinv_block(dega_ref, degb_ref):
    deg = dega_ref[:, 0:1] + degb_ref[:, 0:1] + 1.0
    return lax.rsqrt(deg)


def _mm_body(x_ref, w_ref, o_ref):
    o_ref[...] = jnp.dot(x_ref[...], w_ref[...],
                         preferred_element_type=_f32,
                         precision=lax.Precision.HIGHEST)


def _scale_body(h_ref, dega_ref, degb_ref, o_ref):
    o_ref[...] = h_ref[...] * _dinv_block(dega_ref, degb_ref)


def _mid_body(g_ref, acca_ref, accb_ref, dega_ref, degb_ref, w_ref, b_ref,
              o_ref):
    dinv = _dinv_block(dega_ref, degb_ref)
    h = (g_ref[...] + acca_ref[...] + accb_ref[...]) * dinv + b_ref[...]
    h = jnp.maximum(h, 0.0)
    o_ref[...] = jnp.dot(h, w_ref[...], preferred_element_type=_f32,
                         precision=lax.Precision.HIGHEST) * dinv


def _final_body(g_ref, acca_ref, accb_ref, dega_ref, degb_ref, b_ref, o_ref):
    dinv = _dinv_block(dega_ref, degb_ref)
    o_ref[...] = (g_ref[...] + acca_ref[...] + accb_ref[...]) * dinv + b_ref[...]


_row_spec = pl.BlockSpec((BLK, D), lambda i: (i, 0))
_deg_spec = pl.BlockSpec((BLK, 1), lambda i: (i, 0))
_w_spec = pl.BlockSpec((D, D), lambda i: (0, 0))
_b_spec = pl.BlockSpec((1, D), lambda i: (0, 0))
_row_out = jax.ShapeDtypeStruct((NPAD, D), _f32)

FBLK = 2000
FGRID = N // FBLK
_frow_spec = pl.BlockSpec((FBLK, D), lambda i: (i, 0))
_fdeg_spec = pl.BlockSpec((FBLK, 1), lambda i: (i, 0))
_fb_spec = pl.BlockSpec((1, D), lambda i: (0, 0))

_tc_mm = pl.pallas_call(
    _mm_body, grid=(GRID,), out_shape=_row_out,
    in_specs=[_row_spec, _w_spec], out_specs=_row_spec)

_tc_scale = pl.pallas_call(
    _scale_body, grid=(GRID,), out_shape=_row_out,
    in_specs=[_row_spec, _deg_spec, _deg_spec], out_specs=_row_spec)

_tc_mid = pl.pallas_call(
    _mid_body, grid=(GRID,), out_shape=_row_out,
    in_specs=[_row_spec, _row_spec, _row_spec, _deg_spec, _deg_spec,
              _w_spec, _b_spec],
    out_specs=_row_spec)

_tc_final = pl.pallas_call(
    _final_body, grid=(FGRID,), out_shape=jax.ShapeDtypeStruct((N, D), _f32),
    in_specs=[_frow_spec, _frow_spec, _frow_spec, _fdeg_spec, _fdeg_spec,
              _fb_spec],
    out_specs=_frow_spec)


def kernel(x, edge_index, W1, b1, W2, b2):
    src = edge_index[0].astype(jnp.int32)
    dst = edge_index[1].astype(jnp.int32)
    pad = N + jnp.arange(EPAD - E, dtype=jnp.int32) % (NPAD - N)
    src_r = jnp.concatenate([src, pad]).reshape(NW, CH, CHUNK)
    dst_r = jnp.concatenate([dst, pad]).reshape(NW, CH, CHUNK)
    b1r = b1.reshape(1, D)
    b2r = b2.reshape(1, D)

    deg = _sc_degree(dst_r)
    dega = deg[0].reshape(NPAD, 1)
    degb = deg[1].reshape(NPAD, 1)

    h1 = _tc_mm(x, W1)
    g1 = _tc_scale(h1, dega, degb)
    acc1 = _sc_edge_pass(g1, src_r, dst_r)
    g2 = _tc_mid(g1, acc1[0], acc1[1], dega, degb, W2, b1r)
    acc2 = _sc_edge_pass(g2, src_r, dst_r)
    return _tc_final(g2, acc2[0], acc2[1], dega, degb, b2r)


# speedup vs baseline: 1.4800x; 1.0284x over previous
import dataclasses
import functools

import jax
import jax.numpy as jnp
from jax import lax
from jax.experimental import pallas as pl
from jax.experimental.pallas import tpu as pltpu
from jax.experimental.pallas import tpu_sc as plsc

N = 10000
D = 128
E = 320000

NC = 2
NS = 16
NW = NC * NS
CHUNK = 128
CH = 80
IB = 40
EPAD = NW * CH * CHUNK
NPAD = 10240
RPS = NPAD // NS

BLK = 1280
GRID = NPAD // BLK

_mesh = plsc.VectorSubcoreMesh(core_axis_name="c", subcore_axis_name="s")
_f32 = jnp.float32


def _fill(ref, value):
    @pl.loop(0, ref.shape[0])
    def _(i):
        @pl.loop(0, ref.shape[1], step=16)
        def _(c):
            ref.at[i, pl.ds(c, 16)][...] = jnp.full((16,), value, _f32)


HR = NPAD // CHUNK

_sc_params = pltpu.CompilerParams()
if "needs_layout_passes" in pltpu.CompilerParams.__dataclass_fields__:
    _sc_params = dataclasses.replace(_sc_params, needs_layout_passes=False)


@functools.partial(
    pl.kernel,
    out_type=jax.ShapeDtypeStruct((NC, HR, CHUNK), _f32),
    mesh=_mesh,
    compiler_params=_sc_params,
    scratch_types=[
        pltpu.VMEM((CH, CHUNK), jnp.int32),
        pltpu.VMEM((HR, CHUNK), _f32),
        pltpu.VMEM((8, CHUNK), _f32),
        pltpu.VMEM((HR,), jnp.int32),
        pltpu.VMEM_SHARED((HR, CHUNK), _f32),
    ],
)
def _sc_degree(dst_hbm, out_hbm, dst_v, hist_v, zero_v, iota_v, acc_sh):
    cid = lax.axis_index("c")
    sid = lax.axis_index("s")
    wid = sid * NC + cid

    lane = lax.iota(jnp.int32, 16)

    @pl.loop(0, HR)
    def _(r):
        @pl.loop(0, CHUNK, step=16)
        def _(c):
            hist_v.at[r, pl.ds(c, 16)][...] = jnp.zeros((16,), _f32)

    @pl.loop(0, 8)
    def _(r):
        @pl.loop(0, CHUNK, step=16)
        def _(c):
            zero_v.at[r, pl.ds(c, 16)][...] = jnp.zeros((16,), _f32)

    @pl.loop(0, HR, step=16)
    def _(r):
        iota_v.at[pl.ds(r, 16)][...] = lane + r

    @pl.when(sid < HR // 8)
    def _():
        pltpu.sync_copy(zero_v, acc_sh.at[pl.ds(sid * 8, 8)])

    plsc.subcore_barrier()
    pltpu.sync_copy(dst_hbm.at[wid], dst_v)

    ones = jnp.full((16,), 1.0, _f32)

    @pl.loop(0, CH)
    def _(j):
        @pl.loop(0, CHUNK, step=16)
        def _(c):
            idx = dst_v.at[j, pl.ds(c, 16)][...]
            row = lax.shift_right_logical(idx, 7)
            col = lax.bitwise_and(idx, 127)
            plsc.addupdate_scatter(hist_v, [row, col], ones)

    pltpu.sync_copy(hist_v, acc_sh.at[iota_v], add=True)
    plsc.subcore_barrier()

    @pl.when(sid < HR // 8)
    def _():
        pltpu.sync_copy(acc_sh.at[pl.ds(sid * 8, 8)],
                        out_hbm.at[cid, pl.ds(sid * 8, 8)])


NB = 4
EC = 64
CHE = EPAD // NW // EC
IBE = 40


@functools.partial(
    pl.kernel,
    out_type=jax.ShapeDtypeStruct((NC, NPAD, D), _f32),
    mesh=_mesh,
    scratch_types=[
        pltpu.VMEM((IBE, EC), jnp.int32),
        pltpu.VMEM((IBE, EC), jnp.int32),
        [pltpu.VMEM((EC, D), _f32)] * NB,
        pltpu.VMEM_SHARED((NPAD, D), _f32),
        [pltpu.SemaphoreType.DMA] * NB,
    ],
)
def _sc_edge_pass(g_hbm, src_hbm, dst_hbm, out_hbm,
                  src_i, dst_i, rows, acc_sh, sems):
    cid = lax.axis_index("c")
    sid = lax.axis_index("s")
    wid = sid * NC + cid

    _fill(rows[0], 0.0)

    @pl.loop(0, RPS // EC)
    def _(k):
        pltpu.sync_copy(rows[0], acc_sh.at[pl.ds(sid * RPS + k * EC, EC)])

    plsc.subcore_barrier()

    @pl.loop(0, CHE, step=IBE)
    def _(jb):
        pltpu.sync_copy(src_hbm.at[wid, pl.ds(jb, IBE)], src_i)
        pltpu.sync_copy(dst_hbm.at[wid, pl.ds(jb, IBE)], dst_i)
        for k in range(NB):
            pltpu.async_copy(g_hbm.at[src_i.at[k]], rows[k], sems[k])

        @pl.loop(0, IBE, step=NB)
        def _(t):
            for k in range(NB):
                pltpu.make_async_copy(g_hbm.at[src_i.at[t + k]], rows[k],
                                      sems[k]).wait()
                pltpu.sync_copy(rows[k], acc_sh.at[dst_i.at[t + k]], add=True)

                @pl.when(t + k + NB < IBE)
                def _(k=k):
                    pltpu.async_copy(g_hbm.at[src_i.at[t + k + NB]], rows[k],
                                     sems[k])

    plsc.subcore_barrier()
    pltpu.sync_copy(acc_sh.at[pl.ds(sid * RPS, RPS)],
                    out_hbm.at[cid, pl.ds(sid * RPS, RPS)])


def _dinv_block(dega_ref, degb_ref):
    deg = dega_ref[:, 0:1] + degb_ref[:, 0:1] + 1.0
    return lax.rsqrt(deg)


def _mm_body(x_ref, w_ref, o_ref):
    o_ref[...] = jnp.dot(x_ref[...], w_ref[...],
                         preferred_element_type=_f32,
                         precision=lax.Precision.HIGHEST)


def _scale_body(h_ref, dega_ref, degb_ref, o_ref):
    o_ref[...] = h_ref[...] * _dinv_block(dega_ref, degb_ref)


def _mid_body(g_ref, acca_ref, accb_ref, dega_ref, degb_ref, w_ref, b_ref,
              o_ref):
    dinv = _dinv_block(dega_ref, degb_ref)
    h = (g_ref[...] + acca_ref[...] + accb_ref[...]) * dinv + b_ref[...]
    h = jnp.maximum(h, 0.0)
    o_ref[...] = jnp.dot(h, w_ref[...], preferred_element_type=_f32,
                         precision=lax.Precision.HIGHEST) * dinv


def _final_body(g_ref, acca_ref, accb_ref, dega_ref, degb_ref, b_ref, o_ref):
    dinv = _dinv_block(dega_ref, degb_ref)
    o_ref[...] = (g_ref[...] + acca_ref[...] + accb_ref[...]) * dinv + b_ref[...]


_row_spec = pl.BlockSpec((BLK, D), lambda i: (i, 0))
_deg_spec = pl.BlockSpec((BLK, 1), lambda i: (i, 0))
_w_spec = pl.BlockSpec((D, D), lambda i: (0, 0))
_b_spec = pl.BlockSpec((1, D), lambda i: (0, 0))
_row_out = jax.ShapeDtypeStruct((NPAD, D), _f32)

FBLK = 2000
FGRID = N // FBLK
_frow_spec = pl.BlockSpec((FBLK, D), lambda i: (i, 0))
_fdeg_spec = pl.BlockSpec((FBLK, 1), lambda i: (i, 0))
_fb_spec = pl.BlockSpec((1, D), lambda i: (0, 0))

_tc_mm = pl.pallas_call(
    _mm_body, grid=(GRID,), out_shape=_row_out,
    in_specs=[_row_spec, _w_spec], out_specs=_row_spec)

_tc_scale = pl.pallas_call(
    _scale_body, grid=(GRID,), out_shape=_row_out,
    in_specs=[_row_spec, _deg_spec, _deg_spec], out_specs=_row_spec)

_tc_mid = pl.pallas_call(
    _mid_body, grid=(GRID,), out_shape=_row_out,
    in_specs=[_row_spec, _row_spec, _row_spec, _deg_spec, _deg_spec,
              _w_spec, _b_spec],
    out_specs=_row_spec)

_tc_final = pl.pallas_call(
    _final_body, grid=(FGRID,), out_shape=jax.ShapeDtypeStruct((N, D), _f32),
    in_specs=[_frow_spec, _frow_spec, _frow_spec, _fdeg_spec, _fdeg_spec,
              _fb_spec],
    out_specs=_frow_spec)


def kernel(x, edge_index, W1, b1, W2, b2):
    src = edge_index[0].astype(jnp.int32)
    dst = edge_index[1].astype(jnp.int32)
    pad = N + jnp.arange(EPAD - E, dtype=jnp.int32) % (NPAD - N)
    src_p = jnp.concatenate([src, pad])
    dst_p = jnp.concatenate([dst, pad])
    dst_r = dst_p.reshape(NW, CH, CHUNK)
    src_e = src_p.reshape(NW, CHE, EC)
    dst_e = dst_p.reshape(NW, CHE, EC)
    b1r = b1.reshape(1, D)
    b2r = b2.reshape(1, D)

    deg = _sc_degree(dst_r)
    dega = deg[0].reshape(NPAD, 1)
    degb = deg[1].reshape(NPAD, 1)

    h1 = _tc_mm(x, W1)
    g1 = _tc_scale(h1, dega, degb)
    acc1 = _sc_edge_pass(g1, src_e, dst_e)
    g2 = _tc_mid(g1, acc1[0], acc1[1], dega, degb, W2, b1r)
    acc2 = _sc_edge_pass(g2, src_e, dst_e)
    return _tc_final(g2, acc2[0], acc2[1], dega, degb, b2r)
